# baseline (speedup 1.0000x reference)
import functools

import jax
import jax.numpy as jnp
from jax import lax
from jax.experimental import pallas as pl
from jax.experimental.pallas import tpu as pltpu
from jax.experimental.pallas import tpu_sc as plsc

EMB_DIM = 32
NUM_WORKERS = 32

VBLK = 512
N_BLKS = 1000000 // VBLK
SB_ROWS = VBLK * EMB_DIM // 128
TAIL = 1000000 - N_BLKS * VBLK
TAIL_SB = TAIL * EMB_DIM // 128

NBUF = 8
CHUNK = 200


def _detile(table_t, tail_rm):
    mesh = plsc.VectorSubcoreMesh(core_axis_name="c", subcore_axis_name="s")

    @functools.partial(
        pl.kernel,
        mesh=mesh,
        out_type=jax.ShapeDtypeStruct((N_BLKS * SB_ROWS + TAIL_SB, 128), jnp.float32),
        scratch_types=[
            pltpu.VMEM((2, EMB_DIM, VBLK + 1), jnp.float32),
            pltpu.VMEM((2, SB_ROWS, 128), jnp.float32),
            pltpu.VMEM((TAIL, EMB_DIM), jnp.float32),
            pltpu.SemaphoreType.DMA((2,)),
            pltpu.SemaphoreType.DMA((2,)),
        ],
        compiler_params=pltpu.CompilerParams(
            use_tc_tiling_on_sc=True, needs_layout_passes=False
        ),
    )
    def detile_kernel(tt_hbm, tail_hbm, s_hbm, in_v, sb_v, tail_v, isem, wsem):
        wid = lax.axis_index("s") * 2 + lax.axis_index("c")
        base_n = N_BLKS // NUM_WORKERS
        extra = N_BLKS % NUM_WORKERS
        start = wid * base_n + jnp.minimum(wid, extra)
        n_w = base_n + jnp.where(wid < extra, 1, 0)
        n_pairs = n_w // 2

        e_lo = lax.iota(jnp.int32, 16)
        e_hi = e_lo + 16
        k_const = [jnp.full((16,), kk, jnp.int32) for kk in range(16)]

        def in_copy(i, b):
            return pltpu.make_async_copy(
                tt_hbm.at[:, pl.ds((start + i) * VBLK, VBLK)],
                in_v.at[b].at[:, pl.ds(0, VBLK)],
                isem.at[b],
            )

        def out_copy(i, b):
            return pltpu.make_async_copy(
                sb_v.at[b],
                s_hbm.at[pl.ds((start + i) * SB_ROWS, SB_ROWS)],
                wsem.at[b],
            )

        def transpose_block(b):
            @plsc.parallel_loop(
                0, SB_ROWS, unroll=4, carry=jnp.zeros((16,), jnp.int32)
            )
            def _(s, c_vec):
                for m in range(8):
                    e_idx = e_lo if m % 2 == 0 else e_hi
                    v_idx = c_vec + k_const[m // 2]
                    val = plsc.load_gather(in_v.at[b], [e_idx, v_idx])
                    sb_v[b, s, pl.ds(16 * m, 16)] = val
                return c_vec + k_const[4]

        in_copy(0, 0).start()

        @pl.when(n_w > 1)
        def _():
            in_copy(1, 1).start()

        @pl.loop(0, n_pairs)
        def _(g):
            for b in range(2):
                i = 2 * g + b
                in_copy(i, b).wait()

                @pl.when(g > 0)
                def _():
                    out_copy(0, b).wait()

                transpose_block(b)
                out_copy(i, b).start()

                @pl.when(i + 2 < n_w)
                def _():
                    in_copy(i + 2, b).start()

        @pl.when(n_w % 2 == 1)
        def _():
            i = n_w - 1
            in_copy(i, 0).wait()

            @pl.when(n_pairs > 0)
            def _():
                out_copy(0, 0).wait()

            transpose_block(0)
            pltpu.sync_copy(
                sb_v.at[0], s_hbm.at[pl.ds((start + i) * SB_ROWS, SB_ROWS)]
            )

        @pl.when((n_w % 2 == 0) & (n_pairs > 0))
        def _():
            out_copy(0, 0).wait()

        @pl.when(n_pairs > 0)
        def _():
            out_copy(0, 1).wait()

        @pl.when(wid == NUM_WORKERS - 1)
        def _():
            pltpu.sync_copy(tail_hbm, tail_v)

            @pl.loop(0, TAIL_SB)
            def _(s):
                for m in range(8):
                    e_idx = e_lo if m % 2 == 0 else e_hi
                    v_idx = jnp.full((16,), 4 * s + m // 2, jnp.int32)
                    val = plsc.load_gather(tail_v, [v_idx, e_idx])
                    sb_v[0, s, pl.ds(16 * m, 16)] = val

            pltpu.sync_copy(
                sb_v.at[0].at[pl.ds(0, TAIL_SB)],
                s_hbm.at[pl.ds(N_BLKS * SB_ROWS, TAIL_SB)],
            )

    return detile_kernel(table_t, tail_rm)


def _gather(table_rm, idx):
    n = idx.shape[0]
    b_per_w = n // NUM_WORKERS
    group = NBUF * CHUNK
    n_groups = b_per_w // group
    assert n % NUM_WORKERS == 0 and b_per_w % group == 0
    mesh = plsc.VectorSubcoreMesh(core_axis_name="c", subcore_axis_name="s")

    @functools.partial(
        pl.kernel,
        mesh=mesh,
        out_type=jax.ShapeDtypeStruct((n, EMB_DIM), jnp.float32),
        scratch_types=[
            pltpu.VMEM((b_per_w,), jnp.int32),
            pltpu.VMEM((NBUF, CHUNK, EMB_DIM), jnp.float32),
            pltpu.SemaphoreType.DMA((NBUF,)),
            pltpu.SemaphoreType.DMA((NBUF,)),
        ],
        compiler_params=pltpu.CompilerParams(use_tc_tiling_on_sc=False),
    )
    def gather_kernel(table_hbm, idx_hbm, out_hbm, idx_v, rows_v, gsem, wsem):
        wid = lax.axis_index("s") * 2 + lax.axis_index("c")
        base = wid * b_per_w
        pltpu.sync_copy(idx_hbm.at[pl.ds(base, b_per_w)], idx_v)

        @pl.loop(0, n_groups)
        def _(g):
            goff = g * group

            for b in range(NBUF):
                @pl.when(g > 0)
                def _():
                    pltpu.make_async_copy(
                        rows_v.at[b],
                        out_hbm.at[pl.ds(base + goff - group + b * CHUNK, CHUNK)],
                        wsem.at[b],
                    ).wait()

                pltpu.async_copy(
                    table_hbm.at[idx_v.at[pl.ds(goff + b * CHUNK, CHUNK)]],
                    rows_v.at[b],
                    gsem.at[b],
                )

            for b in range(NBUF):
                pltpu.make_async_copy(
                    table_hbm.at[idx_v.at[pl.ds(goff + b * CHUNK, CHUNK)]],
                    rows_v.at[b],
                    gsem.at[b],
                ).wait()
                pltpu.async_copy(
                    rows_v.at[b],
                    out_hbm.at[pl.ds(base + goff + b * CHUNK, CHUNK)],
                    wsem.at[b],
                )

        goff = (n_groups - 1) * group
        for b in range(NBUF):
            pltpu.make_async_copy(
                rows_v.at[b],
                out_hbm.at[pl.ds(base + goff + b * CHUNK, CHUNK)],
                wsem.at[b],
            ).wait()

    return gather_kernel(table_rm, idx)


def kernel(x, table):
    batch, seq = x.shape
    n = batch * seq
    s = _detile(table.T, table[N_BLKS * VBLK:])
    table_rm = s.reshape(1000000, EMB_DIM)
    out = _gather(table_rm, x.reshape(n))
    return out.reshape(batch, seq, EMB_DIM)


# speedup vs baseline: 1.0002x; 1.0002x over previous
import functools

import jax
import jax.numpy as jnp
from jax import lax
from jax.experimental import pallas as pl
from jax.experimental.pallas import tpu as pltpu
from jax.experimental.pallas import tpu_sc as plsc

EMB_DIM = 32
NUM_WORKERS = 32

VBLK = 512
N_BLKS = 1000000 // VBLK
SB_ROWS = VBLK * EMB_DIM // 128
TAIL = 1000000 - N_BLKS * VBLK
TAIL_SB = TAIL * EMB_DIM // 128

NBUF = 8
CHUNK = 200


def _detile(table_t, tail_rm):
    mesh = plsc.VectorSubcoreMesh(core_axis_name="c", subcore_axis_name="s")

    @functools.partial(
        pl.kernel,
        mesh=mesh,
        out_type=jax.ShapeDtypeStruct((N_BLKS * SB_ROWS + TAIL_SB, 128), jnp.float32),
        scratch_types=[
            pltpu.VMEM((2, EMB_DIM, VBLK + 1), jnp.float32),
            pltpu.VMEM((2, SB_ROWS, 128), jnp.float32),
            pltpu.VMEM((TAIL, EMB_DIM), jnp.float32),
            pltpu.SemaphoreType.DMA((2,)),
            pltpu.SemaphoreType.DMA((2,)),
        ],
        compiler_params=pltpu.CompilerParams(
            use_tc_tiling_on_sc=True, needs_layout_passes=False
        ),
    )
    def detile_kernel(tt_hbm, tail_hbm, s_hbm, in_v, sb_v, tail_v, isem, wsem):
        wid = lax.axis_index("s") * 2 + lax.axis_index("c")
        base_n = N_BLKS // NUM_WORKERS
        extra = N_BLKS % NUM_WORKERS
        start = wid * base_n + jnp.minimum(wid, extra)
        n_w = base_n + jnp.where(wid < extra, 1, 0)
        n_pairs = n_w // 2

        e_lo = lax.iota(jnp.int32, 16)
        e_hi = e_lo + 16
        k_const = [jnp.full((16,), kk, jnp.int32) for kk in range(16)]

        def in_copy(i, b):
            return pltpu.make_async_copy(
                tt_hbm.at[:, pl.ds((start + i) * VBLK, VBLK)],
                in_v.at[b].at[:, pl.ds(0, VBLK)],
                isem.at[b],
            )

        def out_copy(i, b):
            return pltpu.make_async_copy(
                sb_v.at[b],
                s_hbm.at[pl.ds((start + i) * SB_ROWS, SB_ROWS)],
                wsem.at[b],
            )

        def transpose_block(b):
            @plsc.parallel_loop(
                0, SB_ROWS, unroll=4, carry=jnp.zeros((16,), jnp.int32)
            )
            def _(s, c_vec):
                for m in range(8):
                    e_idx = e_lo if m % 2 == 0 else e_hi
                    v_idx = c_vec + k_const[m // 2]
                    val = plsc.load_gather(in_v.at[b], [e_idx, v_idx])
                    sb_v[b, s, pl.ds(16 * m, 16)] = val
                return c_vec + k_const[4]

        in_copy(0, 0).start()

        @pl.when(n_w > 1)
        def _():
            in_copy(1, 1).start()

        @pl.loop(0, n_pairs)
        def _(g):
            for b in range(2):
                i = 2 * g + b
                in_copy(i, b).wait()

                @pl.when(g > 0)
                def _():
                    out_copy(0, b).wait()

                transpose_block(b)
                out_copy(i, b).start()

                @pl.when(i + 2 < n_w)
                def _():
                    in_copy(i + 2, b).start()

        @pl.when(n_w % 2 == 1)
        def _():
            i = n_w - 1
            in_copy(i, 0).wait()

            @pl.when(n_pairs > 0)
            def _():
                out_copy(0, 0).wait()

            transpose_block(0)
            pltpu.sync_copy(
                sb_v.at[0], s_hbm.at[pl.ds((start + i) * SB_ROWS, SB_ROWS)]
            )

        @pl.when((n_w % 2 == 0) & (n_pairs > 0))
        def _():
            out_copy(0, 0).wait()

        @pl.when(n_pairs > 0)
        def _():
            out_copy(0, 1).wait()

        @pl.when(wid == NUM_WORKERS - 1)
        def _():
            pltpu.sync_copy(tail_hbm, tail_v)

            @pl.loop(0, TAIL_SB)
            def _(s):
                for m in range(8):
                    e_idx = e_lo if m % 2 == 0 else e_hi
                    v_idx = jnp.full((16,), 4 * s + m // 2, jnp.int32)
                    val = plsc.load_gather(tail_v, [v_idx, e_idx])
                    sb_v[0, s, pl.ds(16 * m, 16)] = val

            pltpu.sync_copy(
                sb_v.at[0].at[pl.ds(0, TAIL_SB)],
                s_hbm.at[pl.ds(N_BLKS * SB_ROWS, TAIL_SB)],
            )

    return detile_kernel(table_t, tail_rm)


def _gather(table_rm, idx, batch, seq):
    n = idx.shape[0]
    b_per_w = n // NUM_WORKERS
    group = NBUF * CHUNK
    n_groups = b_per_w // group
    assert n % NUM_WORKERS == 0 and b_per_w % group == 0
    mesh = plsc.VectorSubcoreMesh(core_axis_name="c", subcore_axis_name="s")

    @functools.partial(
        pl.kernel,
        mesh=mesh,
        out_type=jax.ShapeDtypeStruct((batch, seq, EMB_DIM), jnp.float32),
        scratch_types=[
            pltpu.VMEM((b_per_w,), jnp.int32),
            pltpu.VMEM((NBUF, CHUNK, EMB_DIM), jnp.float32),
            pltpu.SemaphoreType.DMA((NBUF,)),
            pltpu.SemaphoreType.DMA((NBUF,)),
        ],
        compiler_params=pltpu.CompilerParams(use_tc_tiling_on_sc=False),
    )
    def gather_kernel(table_hbm, idx_hbm, out_hbm, idx_v, rows_v, gsem, wsem):
        wid = lax.axis_index("s") * 2 + lax.axis_index("c")
        base = wid * b_per_w
        pltpu.sync_copy(idx_hbm.at[pl.ds(base, b_per_w)], idx_v)

        @pl.loop(0, n_groups)
        def _(g):
            goff = g * group

            for b in range(NBUF):
                @pl.when(g > 0)
                def _():
                    pltpu.make_async_copy(
                        rows_v.at[b],
                        out_hbm.at[(base + goff - group) // CHUNK + b],
                        wsem.at[b],
                    ).wait()

                pltpu.async_copy(
                    table_hbm.at[idx_v.at[pl.ds(goff + b * CHUNK, CHUNK)]],
                    rows_v.at[b],
                    gsem.at[b],
                )

            for b in range(NBUF):
                pltpu.make_async_copy(
                    table_hbm.at[idx_v.at[pl.ds(goff + b * CHUNK, CHUNK)]],
                    rows_v.at[b],
                    gsem.at[b],
                ).wait()
                pltpu.async_copy(
                    rows_v.at[b],
                    out_hbm.at[(base + goff) // CHUNK + b],
                    wsem.at[b],
                )

        goff = (n_groups - 1) * group
        for b in range(NBUF):
            pltpu.make_async_copy(
                rows_v.at[b],
                out_hbm.at[(base + goff) // CHUNK + b],
                wsem.at[b],
            ).wait()

    return gather_kernel(table_rm, idx)


def kernel(x, table):
    batch, seq = x.shape
    n = batch * seq
    s = _detile(table.T, table[N_BLKS * VBLK:])
    table_rm = s.reshape(1000000, EMB_DIM)
    assert seq == CHUNK
    return _gather(table_rm, x.reshape(n), batch, seq)


# speedup vs baseline: 1.1950x; 1.1947x over previous
import functools

import jax
import jax.numpy as jnp
from jax import lax
from jax.experimental import pallas as pl
from jax.experimental.pallas import tpu as pltpu
from jax.experimental.pallas import tpu_sc as plsc

EMB_DIM = 32
NUM_WORKERS = 32

VBLK = 512
N_BLKS = 1000000 // VBLK
SB_ROWS = VBLK * EMB_DIM // 128
TAIL = 1000000 - N_BLKS * VBLK
TAIL_SB = TAIL * EMB_DIM // 128

UCHUNK = 128


def _detile(table_t, tail_rm):
    mesh = plsc.VectorSubcoreMesh(core_axis_name="c", subcore_axis_name="s")

    @functools.partial(
        pl.kernel,
        mesh=mesh,
        out_type=jax.ShapeDtypeStruct((N_BLKS * SB_ROWS + TAIL_SB, 128), jnp.float32),
        scratch_types=[
            pltpu.VMEM((2, EMB_DIM, VBLK + 1), jnp.float32),
            pltpu.VMEM((2, SB_ROWS, 128), jnp.float32),
            pltpu.VMEM((TAIL, EMB_DIM), jnp.float32),
            pltpu.SemaphoreType.DMA((2,)),
            pltpu.SemaphoreType.DMA((2,)),
        ],
        compiler_params=pltpu.CompilerParams(
            use_tc_tiling_on_sc=True, needs_layout_passes=False
        ),
    )
    def detile_kernel(tt_hbm, tail_hbm, s_hbm, in_v, sb_v, tail_v, isem, wsem):
        wid = lax.axis_index("s") * 2 + lax.axis_index("c")
        base_n = N_BLKS // NUM_WORKERS
        extra = N_BLKS % NUM_WORKERS
        start = wid * base_n + jnp.minimum(wid, extra)
        n_w = base_n + jnp.where(wid < extra, 1, 0)
        n_pairs = n_w // 2

        e_lo = lax.iota(jnp.int32, 16)
        e_hi = e_lo + 16
        k_const = [jnp.full((16,), kk, jnp.int32) for kk in range(16)]

        def in_copy(i, b):
            return pltpu.make_async_copy(
                tt_hbm.at[:, pl.ds((start + i) * VBLK, VBLK)],
                in_v.at[b].at[:, pl.ds(0, VBLK)],
                isem.at[b],
            )

        def out_copy(i, b):
            return pltpu.make_async_copy(
                sb_v.at[b],
                s_hbm.at[pl.ds((start + i) * SB_ROWS, SB_ROWS)],
                wsem.at[b],
            )

        def transpose_block(b):
            @plsc.parallel_loop(
                0, SB_ROWS, unroll=4, carry=jnp.zeros((16,), jnp.int32)
            )
            def _(s, c_vec):
                for m in range(8):
                    e_idx = e_lo if m % 2 == 0 else e_hi
                    v_idx = c_vec + k_const[m // 2]
                    val = plsc.load_gather(in_v.at[b], [e_idx, v_idx])
                    sb_v[b, s, pl.ds(16 * m, 16)] = val
                return c_vec + k_const[4]

        in_copy(0, 0).start()

        @pl.when(n_w > 1)
        def _():
            in_copy(1, 1).start()

        @pl.loop(0, n_pairs)
        def _(g):
            for b in range(2):
                i = 2 * g + b
                in_copy(i, b).wait()

                @pl.when(g > 0)
                def _():
                    out_copy(0, b).wait()

                transpose_block(b)
                out_copy(i, b).start()

                @pl.when(i + 2 < n_w)
                def _():
                    in_copy(i + 2, b).start()

        @pl.when(n_w % 2 == 1)
        def _():
            i = n_w - 1
            in_copy(i, 0).wait()

            @pl.when(n_pairs > 0)
            def _():
                out_copy(0, 0).wait()

            transpose_block(0)
            pltpu.sync_copy(
                sb_v.at[0], s_hbm.at[pl.ds((start + i) * SB_ROWS, SB_ROWS)]
            )

        @pl.when((n_w % 2 == 0) & (n_pairs > 0))
        def _():
            out_copy(0, 0).wait()

        @pl.when(n_pairs > 0)
        def _():
            out_copy(0, 1).wait()

        @pl.when(wid == NUM_WORKERS - 1)
        def _():
            pltpu.sync_copy(tail_hbm, tail_v)

            @pl.loop(0, TAIL_SB)
            def _(s):
                for m in range(8):
                    e_idx = e_lo if m % 2 == 0 else e_hi
                    v_idx = jnp.full((16,), 4 * s + m // 2, jnp.int32)
                    val = plsc.load_gather(tail_v, [v_idx, e_idx])
                    sb_v[0, s, pl.ds(16 * m, 16)] = val

            pltpu.sync_copy(
                sb_v.at[0].at[pl.ds(0, TAIL_SB)],
                s_hbm.at[pl.ds(N_BLKS * SB_ROWS, TAIL_SB)],
            )

    return detile_kernel(table_t, tail_rm)


def _gather(table_rm, idxt, batch, seq):
    n = idxt.shape[0]
    units = n // UCHUNK
    u_per_w = units // NUM_WORKERS
    i_per_w = n // NUM_WORKERS
    n_pairs = u_per_w // 2
    assert units % NUM_WORKERS == 0 and u_per_w % 2 == 0
    mesh = plsc.VectorSubcoreMesh(core_axis_name="c", subcore_axis_name="s")

    @functools.partial(
        pl.kernel,
        mesh=mesh,
        out_type=jax.ShapeDtypeStruct(
            (batch * seq * EMB_DIM // 128, 128), jnp.float32
        ),
        scratch_types=[
            pltpu.VMEM((i_per_w,), jnp.int32),
            pltpu.VMEM((2, UCHUNK, EMB_DIM), jnp.float32),
            pltpu.VMEM((2, EMB_DIM, 128), jnp.float32),
            pltpu.SemaphoreType.DMA((2,)),
            pltpu.SemaphoreType.DMA((2,)),
        ],
        compiler_params=pltpu.CompilerParams(
            use_tc_tiling_on_sc=False, needs_layout_passes=False
        ),
    )
    def gather_kernel(table_hbm, idx_hbm, out_hbm, idx_v, g_v, ob_v, gsem, wsem):
        wid = lax.axis_index("s") * 2 + lax.axis_index("c")
        pltpu.sync_copy(idx_hbm.at[pl.ds(wid * i_per_w, i_per_w)], idx_v)

        l_const = [lax.iota(jnp.int32, 16) + 16 * m for m in range(8)]
        one = jnp.full((16,), 1, jnp.int32)

        def start_gather(u, b):
            pltpu.async_copy(
                table_hbm.at[idx_v.at[pl.ds(u * UCHUNK, UCHUNK)]],
                g_v.at[b],
                gsem.at[b],
            )

        def wait_gather(b):
            pltpu.make_async_copy(
                table_hbm.at[idx_v.at[pl.ds(0, UCHUNK)]], g_v.at[b], gsem.at[b]
            ).wait()

        def out_stripe(u, a, b):
            ug = wid * u_per_w + u
            t = ug // 32
            d = ug - 32 * t
            rb = t * 1024 + a * 256 + d * 8
            return pltpu.make_async_copy(
                ob_v.at[b].at[pl.ds(8 * a, 8)],
                out_hbm.at[pl.ds(rb, 8)],
                wsem.at[b],
            )

        def transpose_unit(b):
            @plsc.parallel_loop(0, EMB_DIM, carry=jnp.zeros((16,), jnp.int32))
            def _(e, esp):
                for m in range(8):
                    val = plsc.load_gather(g_v.at[b], [l_const[m], esp])
                    ob_v[b, e, pl.ds(16 * m, 16)] = val
                return esp + one

        start_gather(0, 0)
        start_gather(1, 1)

        @pl.loop(0, n_pairs)
        def _(g):
            for b in range(2):
                u = 2 * g + b
                wait_gather(b)

                @pl.when(g > 0)
                def _():
                    for a in range(4):
                        out_stripe(0, a, b).wait()

                transpose_unit(b)
                for a in range(4):
                    out_stripe(u, a, b).start()

                @pl.when(u + 2 < u_per_w)
                def _():
                    start_gather(u + 2, b)

        for b in range(2):
            for a in range(4):
                out_stripe(0, a, b).wait()

    return gather_kernel(table_rm, idxt)


def kernel(x, table):
    batch, seq = x.shape
    s = _detile(table.T, table[N_BLKS * VBLK:])
    table_rm = s.reshape(1000000, EMB_DIM)
    raw = _gather(table_rm, x.T.reshape(-1), batch, seq)
    r5 = raw.reshape(seq, 4, batch // 128, 8, 128)
    return r5.transpose(2, 4, 0, 1, 3).reshape(batch, seq, EMB_DIM)


# speedup vs baseline: 1.1995x; 1.0038x over previous
import functools

import jax
import jax.numpy as jnp
from jax import lax
from jax.experimental import pallas as pl
from jax.experimental.pallas import tpu as pltpu
from jax.experimental.pallas import tpu_sc as plsc

EMB_DIM = 32
NUM_WORKERS = 32

VBLK = 512
N_BLKS = 1000000 // VBLK
SB_ROWS = VBLK * EMB_DIM // 128
TAIL = 1000000 - N_BLKS * VBLK
TAIL_SB = TAIL * EMB_DIM // 128

UCHUNK = 128


def _detile(table_t, tail_rm):
    mesh = plsc.VectorSubcoreMesh(core_axis_name="c", subcore_axis_name="s")

    @functools.partial(
        pl.kernel,
        mesh=mesh,
        out_type=jax.ShapeDtypeStruct((N_BLKS * SB_ROWS + TAIL_SB, 128), jnp.float32),
        scratch_types=[
            pltpu.VMEM((2, EMB_DIM, VBLK + 1), jnp.float32),
            pltpu.VMEM((2, SB_ROWS, 128), jnp.float32),
            pltpu.VMEM((TAIL, EMB_DIM), jnp.float32),
            pltpu.SemaphoreType.DMA((2,)),
            pltpu.SemaphoreType.DMA((2,)),
        ],
        compiler_params=pltpu.CompilerParams(
            use_tc_tiling_on_sc=True, needs_layout_passes=False
        ),
    )
    def detile_kernel(tt_hbm, tail_hbm, s_hbm, in_v, sb_v, tail_v, isem, wsem):
        wid = lax.axis_index("s") * 2 + lax.axis_index("c")
        base_n = N_BLKS // NUM_WORKERS
        extra = N_BLKS % NUM_WORKERS
        start = wid * base_n + jnp.minimum(wid, extra)
        n_w = base_n + jnp.where(wid < extra, 1, 0)
        n_pairs = n_w // 2

        e_lo = lax.iota(jnp.int32, 16)
        e_hi = e_lo + 16
        k_const = [jnp.full((16,), kk, jnp.int32) for kk in range(16)]

        def in_copy(i, b):
            return pltpu.make_async_copy(
                tt_hbm.at[:, pl.ds((start + i) * VBLK, VBLK)],
                in_v.at[b].at[:, pl.ds(0, VBLK)],
                isem.at[b],
            )

        def out_copy(i, b):
            return pltpu.make_async_copy(
                sb_v.at[b],
                s_hbm.at[pl.ds((start + i) * SB_ROWS, SB_ROWS)],
                wsem.at[b],
            )

        def transpose_block(b):
            @plsc.parallel_loop(
                0, SB_ROWS, unroll=4, carry=jnp.zeros((16,), jnp.int32)
            )
            def _(s, c_vec):
                for m in range(8):
                    e_idx = e_lo if m % 2 == 0 else e_hi
                    v_idx = c_vec + k_const[m // 2]
                    val = plsc.load_gather(in_v.at[b], [e_idx, v_idx])
                    sb_v[b, s, pl.ds(16 * m, 16)] = val
                return c_vec + k_const[4]

        in_copy(0, 0).start()

        @pl.when(n_w > 1)
        def _():
            in_copy(1, 1).start()

        @pl.loop(0, n_pairs)
        def _(g):
            for b in range(2):
                i = 2 * g + b
                in_copy(i, b).wait()

                @pl.when(g > 0)
                def _():
                    out_copy(0, b).wait()

                transpose_block(b)
                out_copy(i, b).start()

                @pl.when(i + 2 < n_w)
                def _():
                    in_copy(i + 2, b).start()

        @pl.when(n_w % 2 == 1)
        def _():
            i = n_w - 1
            in_copy(i, 0).wait()

            @pl.when(n_pairs > 0)
            def _():
                out_copy(0, 0).wait()

            transpose_block(0)
            pltpu.sync_copy(
                sb_v.at[0], s_hbm.at[pl.ds((start + i) * SB_ROWS, SB_ROWS)]
            )

        @pl.when((n_w % 2 == 0) & (n_pairs > 0))
        def _():
            out_copy(0, 0).wait()

        @pl.when(n_pairs > 0)
        def _():
            out_copy(0, 1).wait()

        @pl.when(wid == NUM_WORKERS - 1)
        def _():
            pltpu.sync_copy(tail_hbm, tail_v)

            @pl.loop(0, TAIL_SB)
            def _(s):
                for m in range(8):
                    e_idx = e_lo if m % 2 == 0 else e_hi
                    v_idx = jnp.full((16,), 4 * s + m // 2, jnp.int32)
                    val = plsc.load_gather(tail_v, [v_idx, e_idx])
                    sb_v[0, s, pl.ds(16 * m, 16)] = val

            pltpu.sync_copy(
                sb_v.at[0].at[pl.ds(0, TAIL_SB)],
                s_hbm.at[pl.ds(N_BLKS * SB_ROWS, TAIL_SB)],
            )

    return detile_kernel(table_t, tail_rm)


def _gather(table_rm, idxt, batch, seq):
    n = idxt.shape[0]
    units = n // UCHUNK
    u_per_w = units // NUM_WORKERS
    i_per_w = n // NUM_WORKERS
    n_pairs = u_per_w // 2
    assert units % NUM_WORKERS == 0 and u_per_w % 2 == 0
    mesh = plsc.VectorSubcoreMesh(core_axis_name="c", subcore_axis_name="s")

    @functools.partial(
        pl.kernel,
        mesh=mesh,
        out_type=jax.ShapeDtypeStruct(
            (batch * seq * EMB_DIM // 128, 128), jnp.float32
        ),
        scratch_types=[
            pltpu.VMEM((i_per_w,), jnp.int32),
            pltpu.VMEM((2, UCHUNK, EMB_DIM), jnp.float32),
            pltpu.VMEM((2, EMB_DIM, 128), jnp.float32),
            pltpu.SemaphoreType.DMA((2,)),
            pltpu.SemaphoreType.DMA((2,)),
        ],
        compiler_params=pltpu.CompilerParams(
            use_tc_tiling_on_sc=False, needs_layout_passes=False
        ),
    )
    def gather_kernel(table_hbm, idx_hbm, out_hbm, idx_v, g_v, ob_v, gsem, wsem):
        wid = lax.axis_index("s") * 2 + lax.axis_index("c")
        pltpu.sync_copy(idx_hbm.at[pl.ds(wid * i_per_w, i_per_w)], idx_v)

        l_const = [lax.iota(jnp.int32, 16) + 16 * m for m in range(8)]
        one = jnp.full((16,), 1, jnp.int32)

        def start_gather(u, b):
            pltpu.async_copy(
                table_hbm.at[idx_v.at[pl.ds(u * UCHUNK, UCHUNK)]],
                g_v.at[b],
                gsem.at[b],
            )

        def wait_gather(b):
            pltpu.make_async_copy(
                table_hbm.at[idx_v.at[pl.ds(0, UCHUNK)]], g_v.at[b], gsem.at[b]
            ).wait()

        def out_stripe(u, a, b):
            ug = wid * u_per_w + u
            t = ug // 32
            d = ug - 32 * t
            rb = t * 1024 + a * 256 + d * 8
            return pltpu.make_async_copy(
                ob_v.at[b].at[pl.ds(8 * a, 8)],
                out_hbm.at[pl.ds(rb, 8)],
                wsem.at[b],
            )

        def transpose_unit(b):
            @plsc.parallel_loop(
                0, EMB_DIM, unroll=4, carry=jnp.zeros((16,), jnp.int32)
            )
            def _(e, esp):
                for m in range(8):
                    val = plsc.load_gather(g_v.at[b], [l_const[m], esp])
                    ob_v[b, e, pl.ds(16 * m, 16)] = val
                return esp + one

        start_gather(0, 0)
        start_gather(1, 1)

        @pl.loop(0, n_pairs)
        def _(g):
            for b in range(2):
                u = 2 * g + b
                wait_gather(b)

                @pl.when(g > 0)
                def _():
                    for a in range(4):
                        out_stripe(0, a, b).wait()

                transpose_unit(b)
                for a in range(4):
                    out_stripe(u, a, b).start()

                @pl.when(u + 2 < u_per_w)
                def _():
                    start_gather(u + 2, b)

        for b in range(2):
            for a in range(4):
                out_stripe(0, a, b).wait()

    return gather_kernel(table_rm, idxt)


def kernel(x, table):
    batch, seq = x.shape
    s = _detile(table.T, table[N_BLKS * VBLK:])
    table_rm = s.reshape(1000000, EMB_DIM)
    raw = _gather(table_rm, x.T.reshape(-1), batch, seq)
    r5 = raw.reshape(seq, 4, batch // 128, 8, 128)
    return r5.transpose(2, 4, 0, 1, 3).reshape(batch, seq, EMB_DIM)
